# baseline (speedup 1.0000x reference)
import functools

import jax
import jax.numpy as jnp
from jax import lax
from jax.experimental import pallas as pl
from jax.experimental.pallas import tpu as pltpu
from jax.experimental.pallas import tpu_sc as plsc

H = 16
Q = 2048
K = 2048
RS = 4096
NCHUNK = RS // 16
GRP = 8


def _bucket16(d):
    rp = d - jnp.full((16,), 2047, jnp.int32)
    pos = jnp.where(rp > 0, jnp.full((16,), 16, jnp.int32),
                    jnp.zeros((16,), jnp.int32))
    a = jnp.abs(rp)
    ac = jnp.maximum(a, jnp.full((16,), 1, jnp.int32))
    bits = lax.bitcast_convert_type(ac.astype(jnp.float32), jnp.int32)
    e = (bits >> 23) - jnp.full((16,), 127, jnp.int32)
    t = (ac * ac >= lax.shift_left(jnp.full((16,), 1, jnp.int32),
                                   2 * e + 1)).astype(jnp.int32)
    bl = jnp.minimum(2 * e + t + jnp.full((16,), 2, jnp.int32),
                     jnp.full((16,), 15, jnp.int32))
    small = a < jnp.full((16,), 8, jnp.int32)
    return jnp.where(small, a, bl) + pos


def _rpe_body(w_hbm, out_hbm, w_v, table_v, sem):
    cid = lax.axis_index("c")
    sid = lax.axis_index("s")
    shift = 15 - sid

    pltpu.sync_copy(w_hbm, w_v)

    def build_chunk(cidx, carry):
        base = pl.multiple_of(cidx * 16, 16)
        d = lax.iota(jnp.int32, 16) + (base + shift)
        bkt = _bucket16(d)
        for h in range(H):
            vals = plsc.load_gather(w_v, [bkt, jnp.full((16,), h, jnp.int32)])
            table_v[h, pl.ds(base, 16)] = vals
        return carry

    lax.fori_loop(0, NCHUNK, build_chunk, 0)

    qbase = sid + 16 * cid * 64

    def issue(i):
        q = qbase + 16 * i
        src = pl.multiple_of(2047 - q - shift, 16)
        return pltpu.make_async_copy(
            table_v.at[:, pl.ds(src, K)], out_hbm.at[0, :, q, :], sem)

    def group(g, carry):
        n0 = g * GRP
        cps = [issue(n0 + b) for b in range(GRP)]
        for cp in cps:
            cp.start()
        for cp in cps:
            cp.wait()
        return carry

    lax.fori_loop(0, 64 // GRP, group, 0)


@jax.jit
def _rpe(weight):
    mesh = plsc.VectorSubcoreMesh(core_axis_name="c", subcore_axis_name="s")
    return pl.kernel(
        _rpe_body,
        out_type=jax.ShapeDtypeStruct((1, H, Q, K), jnp.float32),
        mesh=mesh,
        compiler_params=pltpu.CompilerParams(
            needs_layout_passes=False, use_tc_tiling_on_sc=False),
        scratch_types=[
            pltpu.VMEM((32, H), jnp.float32),
            pltpu.VMEM((H, RS), jnp.float32),
            pltpu.SemaphoreType.DMA,
        ],
    )(weight)


def kernel(query_seq_length, key_seq_length, weight):
    del query_seq_length, key_seq_length
    return _rpe(weight)


# speedup vs baseline: 1.6561x; 1.6561x over previous
import jax
import jax.numpy as jnp
from jax import lax
from jax.experimental import pallas as pl
from jax.experimental.pallas import tpu as pltpu
from jax.experimental.pallas import tpu_sc as plsc

H = 16
Q = 2048
K = 2048
TS = 4096
QQ = Q // 8
KK = K // 128
CHUNK = 8 * K
NB = 4


def _bucket16(d):
    rp = d - jnp.full((16,), 2047, jnp.int32)
    pos = jnp.where(rp > 0, jnp.full((16,), 16, jnp.int32),
                    jnp.zeros((16,), jnp.int32))
    a = jnp.abs(rp)
    ac = jnp.maximum(a, jnp.full((16,), 1, jnp.int32))
    bits = lax.bitcast_convert_type(ac.astype(jnp.float32), jnp.int32)
    e = (bits >> 23) - jnp.full((16,), 127, jnp.int32)
    t = (ac * ac >= lax.shift_left(jnp.full((16,), 1, jnp.int32),
                                   2 * e + 1)).astype(jnp.int32)
    bl = jnp.minimum(2 * e + t + jnp.full((16,), 2, jnp.int32),
                     jnp.full((16,), 15, jnp.int32))
    small = a < jnp.full((16,), 8, jnp.int32)
    return jnp.where(small, a, bl) + pos


def _rpe_body(w_hbm, out_hbm, w_v, table_v, ring_v, sems):
    cid = lax.axis_index("c")
    sid = lax.axis_index("s")
    h = sid

    pltpu.sync_copy(w_hbm, w_v)

    def tbl_chunk(c16, carry):
        base = pl.multiple_of(c16 * 16, 16)
        bkt = _bucket16(lax.iota(jnp.int32, 16) + base)
        table_v[pl.ds(base, 16)] = plsc.load_gather(
            w_v, [bkt, jnp.full((16,), h, jnp.int32)])
        return carry

    lax.fori_loop(0, TS // 16, tbl_chunk, 0)

    iota16 = lax.iota(jnp.int32, 16)

    def build_col(m, b, kk):
        colbase = kk * 1024
        for qr in range(8):
            j0 = 128 * kk - (8 * m + qr) + 2047
            for c2 in range(8):
                vals = plsc.load_gather(table_v, [iota16 + (j0 + 16 * c2)])
                off = pl.multiple_of(colbase + qr * 128 + 16 * c2, 16)
                ring_v[b, pl.ds(off, 16)] = vals
        return kk

    def start_dma(m, b):
        dst = pl.multiple_of(m * CHUNK, CHUNK)
        return pltpu.make_async_copy(
            ring_v.at[b], out_hbm.at[h, pl.ds(dst, CHUNK)], sems[b])

    m0 = cid * QQ // 2

    for b in range(NB):
        m = m0 + b
        lax.fori_loop(0, KK, lambda kk, c, m=m, b=b: build_col(m, b, kk), 0)
        start_dma(m, b).start()

    def outer(t, carry):
        for b in range(NB):
            g = 4 * t + b
            m = m0 + g
            kc = m // 16
            start_dma(m, b).wait()
            for o in (-2, -1, 0, 1):
                kk = jnp.clip(kc + o, 0, KK - 1)
                build_col(m, b, kk)
            start_dma(m, b).start()
        return carry

    lax.fori_loop(1, QQ // 2 // NB, outer, 0)

    for b in range(NB):
        start_dma(m0, b).wait()


@jax.jit
def _rpe(weight):
    mesh = plsc.VectorSubcoreMesh(core_axis_name="c", subcore_axis_name="s")
    flat = pl.kernel(
        _rpe_body,
        out_type=jax.ShapeDtypeStruct((H, Q * K), jnp.float32),
        mesh=mesh,
        compiler_params=pltpu.CompilerParams(
            needs_layout_passes=False, use_tc_tiling_on_sc=False),
        scratch_types=[
            pltpu.VMEM((32, H), jnp.float32),
            pltpu.VMEM((TS,), jnp.float32),
            pltpu.VMEM((NB, CHUNK), jnp.float32),
            [pltpu.SemaphoreType.DMA] * NB,
        ],
    )(weight)
    t5 = flat.reshape(H, QQ, KK, 8, 128)
    return lax.transpose(t5, (0, 1, 3, 2, 4)).reshape(1, H, Q, K)


def kernel(query_seq_length, key_seq_length, weight):
    del query_seq_length, key_seq_length
    return _rpe(weight)


# speedup vs baseline: 2.8706x; 1.7334x over previous
import jax
import jax.numpy as jnp
from jax import lax
from jax.experimental import pallas as pl
from jax.experimental.pallas import tpu as pltpu
from jax.experimental.pallas import tpu_sc as plsc

H = 16
Q = 2048
K = 2048
TS = 4096
QQ = Q // 8
KK = K // 128
CHUNK = 8 * K
NB = 4


def _bucket16(d):
    rp = d - jnp.full((16,), 2047, jnp.int32)
    pos = jnp.where(rp > 0, jnp.full((16,), 16, jnp.int32),
                    jnp.zeros((16,), jnp.int32))
    a = jnp.abs(rp)
    ac = jnp.maximum(a, jnp.full((16,), 1, jnp.int32))
    bits = lax.bitcast_convert_type(ac.astype(jnp.float32), jnp.int32)
    e = (bits >> 23) - jnp.full((16,), 127, jnp.int32)
    t = (ac * ac >= lax.shift_left(jnp.full((16,), 1, jnp.int32),
                                   2 * e + 1)).astype(jnp.int32)
    bl = jnp.minimum(2 * e + t + jnp.full((16,), 2, jnp.int32),
                     jnp.full((16,), 15, jnp.int32))
    small = a < jnp.full((16,), 8, jnp.int32)
    return jnp.where(small, a, bl) + pos


def _rpe_body(w_hbm, out_hbm, w_v, table_v, ring_v, sems):
    cid = lax.axis_index("c")
    sid = lax.axis_index("s")
    h = sid

    pltpu.sync_copy(w_hbm, w_v)

    def tbl_chunk(c16, carry):
        base = pl.multiple_of(c16 * 16, 16)
        bkt = _bucket16(lax.iota(jnp.int32, 16) + base)
        table_v[pl.ds(base, 16)] = plsc.load_gather(
            w_v, [bkt, jnp.full((16,), h, jnp.int32)])
        return carry

    lax.fori_loop(0, TS // 16, tbl_chunk, 0)

    iota16 = lax.iota(jnp.int32, 16)

    def build_col(m, b, kk):
        colbase = kk * 1024
        base = iota16 + (128 * kk - 8 * m + 2047)
        for half in range(2):
            qrs = range(4 * half, 4 * half + 4)
            vals = [plsc.load_gather(table_v, [base + (16 * c2 - qr)])
                    for qr in qrs for c2 in range(8)]
            i = 0
            for qr in qrs:
                for c2 in range(8):
                    off = pl.multiple_of(colbase + qr * 128 + 16 * c2, 16)
                    ring_v[b, pl.ds(off, 16)] = vals[i]
                    i += 1
        return kk

    def start_dma(m, b):
        dst = pl.multiple_of(m * CHUNK, CHUNK)
        return pltpu.make_async_copy(
            ring_v.at[b], out_hbm.at[h, pl.ds(dst, CHUNK)], sems[b])

    m0 = cid * QQ // 2

    for b in range(NB):
        m = m0 + b
        lax.fori_loop(0, KK, lambda kk, c, m=m, b=b: build_col(m, b, kk), 0)
        start_dma(m, b).start()

    def outer(t, carry):
        for b in range(NB):
            g = 4 * t + b
            m = m0 + g
            kc = m // 16
            start_dma(m, b).wait()
            for o in (-2, -1, 0, 1):
                kk = jnp.clip(kc + o, 0, KK - 1)
                build_col(m, b, kk)
            start_dma(m, b).start()
        return carry

    lax.fori_loop(1, QQ // 2 // NB, outer, 0)

    for b in range(NB):
        start_dma(m0, b).wait()


@jax.jit
def _rpe(weight):
    mesh = plsc.VectorSubcoreMesh(core_axis_name="c", subcore_axis_name="s")
    flat = pl.kernel(
        _rpe_body,
        out_type=jax.ShapeDtypeStruct((H, Q * K), jnp.float32),
        mesh=mesh,
        compiler_params=pltpu.CompilerParams(
            needs_layout_passes=False, use_tc_tiling_on_sc=False),
        scratch_types=[
            pltpu.VMEM((32, H), jnp.float32),
            pltpu.VMEM((TS,), jnp.float32),
            pltpu.VMEM((NB, CHUNK), jnp.float32),
            [pltpu.SemaphoreType.DMA] * NB,
        ],
    )(weight)
    t5 = flat.reshape(H, QQ, KK, 8, 128)
    return lax.transpose(t5, (0, 1, 3, 2, 4)).reshape(1, H, Q, K)


def kernel(query_seq_length, key_seq_length, weight):
    del query_seq_length, key_seq_length
    return _rpe(weight)
